# baseline (speedup 1.0000x reference)
import functools
import math

import jax
import jax.numpy as jnp
from jax.experimental import pallas as pl
from jax.experimental.pallas import tpu as pltpu

_EPS = 1e-5


def _layernorm(x, g, b):
    mu = x.mean(axis=-1, keepdims=True)
    var = ((x - mu) ** 2).mean(axis=-1, keepdims=True)
    return (x - mu) * jax.lax.rsqrt(var + _EPS) * g + b


def _gelu_exact(x):
    return 0.5 * x * (1.0 + jax.lax.erf(x * (1.0 / math.sqrt(2.0))))


def _mlp_quant_chain(pooled, ew1, eb1, eg, ebeta, ew2, eb2,
                     dw1, db1, dg, dbeta, dw2, db2, edges_ref, n_edges):
    hi = jax.lax.Precision.HIGHEST
    h = jnp.dot(pooled, ew1, preferred_element_type=jnp.float32,
                precision=hi) + eb1
    h = _layernorm(h, eg, ebeta)
    h = _gelu_exact(h)
    comm = jnp.dot(h, ew2, preferred_element_type=jnp.float32,
                   precision=hi) + eb2
    edges = [edges_ref[i] for i in range(n_edges)]
    centers = [(edges[i] + edges[i + 1]) * 0.5 for i in range(n_edges - 1)]
    s = jnp.zeros_like(comm)
    for i in range(n_edges):
        s += (comm > edges[i]).astype(jnp.float32)
    xq = jnp.zeros_like(comm) + centers[0]
    for i in range(1, n_edges - 1):
        xq += (s > (i + 0.5)).astype(jnp.float32) * (centers[i] - centers[i - 1])
    xq = jnp.where(comm <= edges[0], edges[0], xq)
    xq = jnp.where(comm > edges[-1], edges[-1], xq)
    h2 = jnp.dot(xq, dw1, preferred_element_type=jnp.float32,
                 precision=hi) + db1
    h2 = _layernorm(h2, dg, dbeta)
    h2 = _gelu_exact(h2)
    return jnp.dot(h2, dw2, preferred_element_type=jnp.float32,
                   precision=hi) + db2


def _fused_kernel(n_batches, n_chunks, n_edges,
                  xa_ref, xb_ref, ew1_ref, eb1_ref, eg_ref, ebeta_ref,
                  ew2_ref, eb2_ref, dw1_ref, db1_ref, dg_ref, dbeta_ref,
                  dw2_ref, db2_ref, edges_ref, out_ref, acc_ref, evec_ref):
    k = pl.program_id(0)
    j = pl.program_id(1)

    @pl.when((k == 0) & (j == 0))
    def _():
        acc_ref[...] = jnp.zeros_like(acc_ref)

    @pl.when(k < n_batches)
    def _():
        kc = jnp.minimum(k, n_batches - 1)
        acc_ref[pl.ds(kc, 1), :] += jnp.sum(xb_ref[0], axis=0, keepdims=True)

    @pl.when((k >= 1) & (j == 0))
    def _():
        seq = xb_ref.shape[1] * n_chunks
        kp = jnp.maximum(k - 1, 0)
        pooled = acc_ref[pl.ds(kp, 1), :] * (1.0 / seq)
        evec_ref[...] = _mlp_quant_chain(
            pooled, ew1_ref[...], eb1_ref[...], eg_ref[...], ebeta_ref[...],
            ew2_ref[...], eb2_ref[...], dw1_ref[...], db1_ref[...],
            dg_ref[...], dbeta_ref[...], dw2_ref[...], db2_ref[...],
            edges_ref, n_edges)

    @pl.when(k >= 1)
    def _():
        out_ref[...] = xa_ref[...] + 0.1 * evec_ref[...][None]


def kernel(hidden_states, enc_w1, enc_b1, enc_g, enc_beta, enc_w2, enc_b2,
           dec_w1, dec_b1, dec_g, dec_beta, dec_w2, dec_b2, bin_edges,
           interpret=False):
    B, S, H = hidden_states.shape
    CH = 512
    n_chunks = S // CH
    n_edges = bin_edges.shape[0]

    row = lambda v: v.reshape(1, -1)
    full2d = lambda a: pl.BlockSpec(a.shape, lambda k, j: (0, 0))

    xa_spec = pl.BlockSpec(
        (1, CH, H),
        lambda k, j: (jnp.maximum(k - 1, 0), jnp.where(k == 0, 0, j), 0))
    xb_spec = pl.BlockSpec(
        (1, CH, H),
        lambda k, j: (jnp.minimum(k, B - 1),
                      jnp.where(k >= B, n_chunks - 1, j), 0))
    out_spec = pl.BlockSpec(
        (1, CH, H),
        lambda k, j: (jnp.maximum(k - 1, 0), jnp.where(k == 0, 0, j), 0))

    out = pl.pallas_call(
        functools.partial(_fused_kernel, B, n_chunks, n_edges),
        out_shape=jax.ShapeDtypeStruct((B, S, H), jnp.float32),
        grid=(B + 1, n_chunks),
        in_specs=[
            xa_spec, xb_spec,
            full2d(enc_w1),
            full2d(row(enc_b1)), full2d(row(enc_g)), full2d(row(enc_beta)),
            full2d(enc_w2), full2d(row(enc_b2)),
            full2d(dec_w1),
            full2d(row(dec_b1)), full2d(row(dec_g)), full2d(row(dec_beta)),
            full2d(dec_w2), full2d(row(dec_b2)),
            pl.BlockSpec(memory_space=pltpu.SMEM),
        ],
        out_specs=out_spec,
        scratch_shapes=[
            pltpu.VMEM((B, H), jnp.float32),
            pltpu.VMEM((1, H), jnp.float32),
        ],
        compiler_params=pltpu.CompilerParams(
            dimension_semantics=("arbitrary", "arbitrary"),
            vmem_limit_bytes=56 * 1024 * 1024,
        ),
        name="vector_comm_fused",
        interpret=interpret,
    )(hidden_states, hidden_states, enc_w1,
      row(enc_b1), row(enc_g), row(enc_beta), enc_w2, row(enc_b2),
      dec_w1, row(dec_b1), row(dec_g), row(dec_beta), dec_w2, row(dec_b2),
      bin_edges)

    return out


# speedup vs baseline: 1.4764x; 1.4764x over previous
import functools
import math

import jax
import jax.numpy as jnp
from jax.experimental import pallas as pl
from jax.experimental.pallas import tpu as pltpu

_EPS = 1e-5
_N_SLABS = 6


def _layernorm(x, g, b):
    mu = x.mean(axis=-1, keepdims=True)
    var = ((x - mu) ** 2).mean(axis=-1, keepdims=True)
    return (x - mu) * jax.lax.rsqrt(var + _EPS) * g + b


def _gelu_exact(x):
    return 0.5 * x * (1.0 + jax.lax.erf(x * (1.0 / math.sqrt(2.0))))


def _mlp_quant_chain(pooled, ew1, eb1, eg, ebeta, ew2, eb2,
                     dw1, db1, dg, dbeta, dw2, db2, edges_ref, n_edges):
    hi = jax.lax.Precision.HIGHEST
    h = jnp.dot(pooled, ew1, preferred_element_type=jnp.float32,
                precision=hi) + eb1
    h = _layernorm(h, eg, ebeta)
    h = _gelu_exact(h)
    comm = jnp.dot(h, ew2, preferred_element_type=jnp.float32,
                   precision=hi) + eb2
    edges = [edges_ref[i] for i in range(n_edges)]
    centers = [(edges[i] + edges[i + 1]) * 0.5 for i in range(n_edges - 1)]
    s = jnp.zeros_like(comm)
    for i in range(n_edges):
        s += (comm > edges[i]).astype(jnp.float32)
    xq = jnp.zeros_like(comm) + centers[0]
    for i in range(1, n_edges - 1):
        xq += (s > (i + 0.5)).astype(jnp.float32) * (centers[i] - centers[i - 1])
    xq = jnp.where(comm <= edges[0], edges[0], xq)
    xq = jnp.where(comm > edges[-1], edges[-1], xq)
    h2 = jnp.dot(xq, dw1, preferred_element_type=jnp.float32,
                 precision=hi) + db1
    h2 = _layernorm(h2, dg, dbeta)
    h2 = _gelu_exact(h2)
    return jnp.dot(h2, dw2, preferred_element_type=jnp.float32,
                   precision=hi) + db2


def _fused_kernel(n_total, per_batch, seq, n_edges,
                  x_ref, ew1_ref, eb1_ref, eg_ref, ebeta_ref, ew2_ref,
                  eb2_ref, dw1_ref, db1_ref, dg_ref, dbeta_ref, dw2_ref,
                  db2_ref, edges_ref, out_ref, slabs, acc_ref, evec_ref,
                  rsem, wsem):

    pltpu.make_async_copy(x_ref.at[0], slabs.at[0], rsem.at[0]).start()
    pltpu.make_async_copy(x_ref.at[1], slabs.at[1], rsem.at[1]).start()

    def body(slot, carry):
        s = jax.lax.rem(slot, _N_SLABS)

        @pl.when(slot + 2 < n_total)
        def _():
            s2 = jax.lax.rem(slot + 2, _N_SLABS)

            @pl.when(slot + 2 >= _N_SLABS)
            def _():
                pltpu.make_async_copy(
                    slabs.at[s2],
                    out_ref.at[slot + 2 - _N_SLABS],
                    wsem.at[s2]).wait()

            pltpu.make_async_copy(
                x_ref.at[slot + 2], slabs.at[s2], rsem.at[s2]).start()

        pltpu.make_async_copy(x_ref.at[slot], slabs.at[s], rsem.at[s]).wait()

        @pl.when(jax.lax.rem(slot, per_batch) == 0)
        def _():
            acc_ref[...] = jnp.zeros_like(acc_ref)

        acc_ref[...] += jnp.sum(slabs[s], axis=0, keepdims=True)

        @pl.when(jax.lax.rem(slot, per_batch) == per_batch - 1)
        def _():
            pooled = acc_ref[...] * (1.0 / seq)
            evec_ref[...] = _mlp_quant_chain(
                pooled, ew1_ref[...], eb1_ref[...], eg_ref[...],
                ebeta_ref[...], ew2_ref[...], eb2_ref[...], dw1_ref[...],
                db1_ref[...], dg_ref[...], dbeta_ref[...], dw2_ref[...],
                db2_ref[...], edges_ref, n_edges)
            for c in range(per_batch):
                chunk = slot - (per_batch - 1) + c
                sc = jax.lax.rem(chunk, _N_SLABS)
                slabs[sc] = slabs[sc] + 0.1 * evec_ref[...]
                pltpu.make_async_copy(
                    slabs.at[sc], out_ref.at[chunk], wsem.at[sc]).start()

        return carry

    jax.lax.fori_loop(0, n_total, body, 0)

    for c in range(n_total - _N_SLABS, n_total):
        pltpu.make_async_copy(
            slabs.at[c % _N_SLABS], out_ref.at[c],
            wsem.at[c % _N_SLABS]).wait()


def kernel(hidden_states, enc_w1, enc_b1, enc_g, enc_beta, enc_w2, enc_b2,
           dec_w1, dec_b1, dec_g, dec_beta, dec_w2, dec_b2, bin_edges,
           interpret=False):
    B, S, H = hidden_states.shape
    CH = 1024
    per_batch = S // CH
    n_total = B * per_batch
    n_edges = bin_edges.shape[0]

    row = lambda v: v.reshape(1, -1)
    vmem = lambda: pl.BlockSpec(memory_space=pltpu.VMEM)

    x_chunks = hidden_states.reshape(n_total, CH, H)

    out = pl.pallas_call(
        functools.partial(_fused_kernel, n_total, per_batch, S, n_edges),
        out_shape=jax.ShapeDtypeStruct((n_total, CH, H), jnp.float32),
        in_specs=[
            pl.BlockSpec(memory_space=pl.ANY),
            vmem(),
            vmem(), vmem(), vmem(),
            vmem(), vmem(),
            vmem(),
            vmem(), vmem(), vmem(),
            vmem(), vmem(),
            pl.BlockSpec(memory_space=pltpu.SMEM),
        ],
        out_specs=pl.BlockSpec(memory_space=pl.ANY),
        scratch_shapes=[
            pltpu.VMEM((_N_SLABS, CH, H), jnp.float32),
            pltpu.VMEM((1, H), jnp.float32),
            pltpu.VMEM((1, H), jnp.float32),
            pltpu.SemaphoreType.DMA((_N_SLABS,)),
            pltpu.SemaphoreType.DMA((_N_SLABS,)),
        ],
        compiler_params=pltpu.CompilerParams(
            vmem_limit_bytes=56 * 1024 * 1024,
        ),
        name="vector_comm_manual",
        interpret=interpret,
    )(x_chunks, enc_w1,
      row(enc_b1), row(enc_g), row(enc_beta), enc_w2, row(enc_b2),
      dec_w1, row(dec_b1), row(dec_g), row(dec_beta), dec_w2, row(dec_b2),
      bin_edges)

    return out.reshape(B, S, H)
